# Initial kernel scaffold; baseline (speedup 1.0000x reference)
import jax
import jax.numpy as jnp
from jax.experimental import pallas as pl


def kernel(x, edge_index, W1, b1, W2, b2, Wf, bf):
    raise NotImplementedError("write your pallas kernel here")


# speedup vs baseline: 37.1137x; 37.1137x over previous
import functools

import jax
import jax.numpy as jnp
from jax import lax
from jax.experimental import pallas as pl
from jax.experimental.pallas import tpu as pltpu
from jax.experimental.pallas import tpu_sc as plsc

F32 = jnp.float32

NC = 2
NS = 16
NW = NC * NS
SUB = 128
NSUB = 8
CH = SUB * NSUB
FW = 8
BN = 1024


def _ceil_to(a, m):
    return (a + m - 1) // m * m


def _sc_mesh():
    return plsc.VectorSubcoreMesh(core_axis_name="c", subcore_axis_name="s")


def _deg_body(n_pad, ept, row2d, zeros1, out, acc, idx, ones_v, sem_s):
    c = lax.axis_index("c")
    s = lax.axis_index("s")
    wid = c * NS + s
    for i in range(SUB // 16):
        ones_v[pl.ds(i * 16, 16)] = jnp.ones((16,), F32)
    rpt = n_pad // NS
    pltpu.sync_copy(zeros1.at[pl.ds(s * rpt, rpt)], acc.at[pl.ds(s * rpt, rpt)])
    plsc.subcore_barrier()
    base = wid * (ept // SUB)

    @pl.loop(0, ept // CH)
    def _(k):
        pltpu.sync_copy(row2d.at[pl.ds(base + k * NSUB, NSUB)], idx)
        ds = [pltpu.async_copy(ones_v, acc.at[idx.at[j]], sem_s, add=True)
              for j in range(NSUB)]
        for d in ds:
            d.wait()

    plsc.subcore_barrier()
    pltpu.sync_copy(acc.at[pl.ds(s * rpt, rpt)], out.at[c, pl.ds(s * rpt, rpt)])


def _prop_body(n_pad, ept, u_hbm, row2d, col2d, zeros8, out,
               acc, idx_r, idx_c, gbuf, sem_g, sem_s):
    c = lax.axis_index("c")
    s = lax.axis_index("s")
    wid = c * NS + s
    rpt = n_pad // NS
    pltpu.sync_copy(zeros8.at[pl.ds(s * rpt, rpt)], acc.at[pl.ds(s * rpt, rpt)])
    plsc.subcore_barrier()
    base = wid * (ept // SUB)

    @pl.loop(0, ept // CH)
    def _(k):
        off = base + k * NSUB
        pltpu.sync_copy(row2d.at[pl.ds(off, NSUB)], idx_r)
        pltpu.sync_copy(col2d.at[pl.ds(off, NSUB)], idx_c)
        gd = [pltpu.async_copy(u_hbm.at[idx_r.at[j]],
                               gbuf.at[pl.ds(j * SUB, SUB)], sem_g)
              for j in range(NSUB)]
        for d in gd:
            d.wait()
        sd = [pltpu.async_copy(gbuf.at[pl.ds(j * SUB, SUB)],
                               acc.at[idx_c.at[j]], sem_s, add=True)
              for j in range(NSUB)]
        for d in sd:
            d.wait()

    plsc.subcore_barrier()
    pltpu.sync_copy(acc.at[pl.ds(s * rpt, rpt)], out.at[c, pl.ds(s * rpt, rpt)])


def _tc_a(degp_ref, x_ref, u_ref):
    deg = degp_ref[0] + degp_ref[1] + 1.0
    dis = lax.rsqrt(deg)
    xb = x_ref[...]
    ones = jnp.ones((xb.shape[0], 1), F32)
    zeros = jnp.zeros((xb.shape[0], FW - 5), F32)
    u_ref[...] = dis * jnp.concatenate([xb, ones, zeros], axis=1)


def _tc_b(raw1_ref, u_ref, w1_ref, b1_ref, w2_ref, wf_ref, u2_ref):
    u = u_ref[...]
    rawsum = raw1_ref[0] + raw1_ref[1] + u
    dis = u[:, 4:5]
    out1 = dis * rawsum
    sc = out1[:, 4:5]
    pre = lax.dot_general(out1[:, :4], w1_ref[...], (((1,), (1,)), ((), ())),
                          preferred_element_type=F32) + sc * b1_ref[...]
    h = jnp.maximum(pre, 0.0)
    t = lax.dot_general(h, w2_ref[...], (((1,), (1,)), ((), ())),
                        preferred_element_type=F32)
    z = lax.dot_general(t, wf_ref[...], (((1,), (1,)), ((), ())),
                        preferred_element_type=F32)
    zeros = jnp.zeros((z.shape[0], FW - 4), F32)
    u2_ref[...] = jnp.concatenate([dis * z, sc, dis, zeros], axis=1)


def _tc_c(raw2_ref, u2_ref, wf_ref, b2_ref, bf_ref, out_ref):
    u2 = u2_ref[...]
    dis = u2[:, 3:4]
    sc = u2[:, 2:3]
    rawsum = raw2_ref[0][:, :2] + raw2_ref[1][:, :2] + u2[:, :2]
    cv = lax.dot_general(b2_ref[...], wf_ref[...], (((1,), (1,)), ((), ())),
                         preferred_element_type=F32)
    out_ref[...] = dis * rawsum + sc * cv + bf_ref[...]


def kernel(x, edge_index, W1, b1, W2, b2, Wf, bf):
    n = x.shape[0]
    e = edge_index.shape[1]
    n_pad = _ceil_to(n + 1, max(BN, NS))
    ept = _ceil_to(-(-e // NW), CH)
    e_pad = ept * NW

    ei = edge_index.astype(jnp.int32)
    pad = jnp.full((e_pad - e,), n, dtype=jnp.int32)
    row2d = jnp.concatenate([ei[0], pad]).reshape(e_pad // SUB, SUB)
    col2d = jnp.concatenate([ei[1], pad]).reshape(e_pad // SUB, SUB)
    xpad = jnp.zeros((n_pad, 4), F32).at[:n].set(x)
    zeros1 = jnp.zeros((n_pad,), F32)
    zeros8 = jnp.zeros((n_pad, FW), F32)

    mesh = _sc_mesh()
    sc_params = pltpu.CompilerParams(use_tc_tiling_on_sc=False)

    deg_call = pl.kernel(
        functools.partial(_deg_body, n_pad, ept),
        out_type=jax.ShapeDtypeStruct((NC, n_pad), F32),
        mesh=mesh,
        compiler_params=sc_params,
        scratch_types=[
            pltpu.VMEM_SHARED((n_pad,), F32),
            pltpu.VMEM((NSUB, SUB), jnp.int32),
            pltpu.VMEM((SUB,), F32),
            pltpu.SemaphoreType.DMA,
        ],
    )
    degp = deg_call(row2d, zeros1)

    prop_call = pl.kernel(
        functools.partial(_prop_body, n_pad, ept),
        out_type=jax.ShapeDtypeStruct((NC, n_pad, FW), F32),
        mesh=mesh,
        compiler_params=sc_params,
        scratch_types=[
            pltpu.VMEM_SHARED((n_pad, FW), F32),
            pltpu.VMEM((NSUB, SUB), jnp.int32),
            pltpu.VMEM((NSUB, SUB), jnp.int32),
            pltpu.VMEM((CH, FW), F32),
            pltpu.SemaphoreType.DMA,
            pltpu.SemaphoreType.DMA,
        ],
    )

    nb = n_pad // BN
    u = pl.pallas_call(
        _tc_a,
        grid=(nb,),
        in_specs=[
            pl.BlockSpec((NC, BN, 1), lambda i: (0, i, 0)),
            pl.BlockSpec((BN, 4), lambda i: (i, 0)),
        ],
        out_specs=pl.BlockSpec((BN, FW), lambda i: (i, 0)),
        out_shape=jax.ShapeDtypeStruct((n_pad, FW), F32),
    )(degp.reshape(NC, n_pad, 1), xpad)

    raw1 = prop_call(u, row2d, col2d, zeros8)

    wspec = lambda shp: pl.BlockSpec(shp, lambda i: tuple(0 for _ in shp))
    u2 = pl.pallas_call(
        _tc_b,
        grid=(nb,),
        in_specs=[
            pl.BlockSpec((NC, BN, FW), lambda i: (0, i, 0)),
            pl.BlockSpec((BN, FW), lambda i: (i, 0)),
            wspec((128, 4)),
            wspec((1, 128)),
            wspec((128, 128)),
            wspec((2, 128)),
        ],
        out_specs=pl.BlockSpec((BN, FW), lambda i: (i, 0)),
        out_shape=jax.ShapeDtypeStruct((n_pad, FW), F32),
    )(raw1, u, W1, b1.reshape(1, 128), W2, Wf)

    raw2 = prop_call(u2, row2d, col2d, zeros8)

    outp = pl.pallas_call(
        _tc_c,
        grid=(nb,),
        in_specs=[
            pl.BlockSpec((NC, BN, FW), lambda i: (0, i, 0)),
            pl.BlockSpec((BN, FW), lambda i: (i, 0)),
            wspec((2, 128)),
            wspec((1, 128)),
            wspec((1, 2)),
        ],
        out_specs=pl.BlockSpec((BN, 2), lambda i: (i, 0)),
        out_shape=jax.ShapeDtypeStruct((n_pad, 2), F32),
    )(raw2, u2, Wf, b2.reshape(1, 128), bf.reshape(1, 2))

    return outp[:n]


# speedup vs baseline: 38.2566x; 1.0308x over previous
import functools

import jax
import jax.numpy as jnp
from jax import lax
from jax.experimental import pallas as pl
from jax.experimental.pallas import tpu as pltpu
from jax.experimental.pallas import tpu_sc as plsc

F32 = jnp.float32

NC = 2
NS = 16
NW = NC * NS
SUB = 128
NSUB = 8
CH = SUB * NSUB
FW = 8
FW2 = 2
BN = 4096


def _ceil_to(a, m):
    return (a + m - 1) // m * m


def _sc_mesh():
    return plsc.VectorSubcoreMesh(core_axis_name="c", subcore_axis_name="s")


def _deg_body(n_pad, ept, row2d, zeros1, out, acc, idx, ones_v, sem_s):
    c = lax.axis_index("c")
    s = lax.axis_index("s")
    wid = c * NS + s
    for i in range(SUB // 16):
        ones_v[pl.ds(i * 16, 16)] = jnp.ones((16,), F32)
    rpt = n_pad // NS
    pltpu.sync_copy(zeros1.at[pl.ds(s * rpt, rpt)], acc.at[pl.ds(s * rpt, rpt)])
    plsc.subcore_barrier()
    base = wid * (ept // SUB)

    @pl.loop(0, ept // CH)
    def _(k):
        pltpu.sync_copy(row2d.at[pl.ds(base + k * NSUB, NSUB)], idx)
        ds = [pltpu.async_copy(ones_v, acc.at[idx.at[j]], sem_s, add=True)
              for j in range(NSUB)]
        for d in ds:
            d.wait()

    plsc.subcore_barrier()
    pltpu.sync_copy(acc.at[pl.ds(s * rpt, rpt)], out.at[c, pl.ds(s * rpt, rpt)])


def _prop_body(n_pad, ept, u_hbm, row2d, col2d, zeros8, out,
               acc, idx_r, idx_c, gbuf, sem_g, sem_s):
    c = lax.axis_index("c")
    s = lax.axis_index("s")
    wid = c * NS + s
    rpt = n_pad // NS
    pltpu.sync_copy(zeros8.at[pl.ds(s * rpt, rpt)], acc.at[pl.ds(s * rpt, rpt)])
    plsc.subcore_barrier()
    base = wid * (ept // SUB)

    @pl.loop(0, ept // CH)
    def _(k):
        off = base + k * NSUB
        pltpu.sync_copy(row2d.at[pl.ds(off, NSUB)], idx_r)
        pltpu.sync_copy(col2d.at[pl.ds(off, NSUB)], idx_c)
        gd = [pltpu.async_copy(u_hbm.at[idx_r.at[j]],
                               gbuf.at[pl.ds(j * SUB, SUB)], sem_g)
              for j in range(NSUB)]
        for d in gd:
            d.wait()
        sd = [pltpu.async_copy(gbuf.at[pl.ds(j * SUB, SUB)],
                               acc.at[idx_c.at[j]], sem_s, add=True)
              for j in range(NSUB)]
        for d in sd:
            d.wait()

    plsc.subcore_barrier()
    pltpu.sync_copy(acc.at[pl.ds(s * rpt, rpt)], out.at[c, pl.ds(s * rpt, rpt)])


def _tc_a(degp_ref, x_ref, u_ref):
    deg = degp_ref[0] + degp_ref[1] + 1.0
    dis = lax.rsqrt(deg)
    xb = x_ref[...]
    ones = jnp.ones((xb.shape[0], 1), F32)
    zeros = jnp.zeros((xb.shape[0], FW - 5), F32)
    u_ref[...] = dis * jnp.concatenate([xb, ones, zeros], axis=1)


def _tc_b(raw1_ref, u_ref, w1_ref, b1_ref, w2_ref, wf_ref, u2_ref, sd_ref):
    u = u_ref[...]
    rawsum = raw1_ref[0] + raw1_ref[1] + u
    dis = u[:, 4:5]
    out1 = dis * rawsum
    sc = out1[:, 4:5]
    pre = lax.dot_general(out1[:, :4], w1_ref[...], (((1,), (1,)), ((), ())),
                          preferred_element_type=F32) + sc * b1_ref[...]
    h = jnp.maximum(pre, 0.0)
    t = lax.dot_general(h, w2_ref[...], (((1,), (1,)), ((), ())),
                        preferred_element_type=F32)
    z = lax.dot_general(t, wf_ref[...], (((1,), (1,)), ((), ())),
                        preferred_element_type=F32)
    u2_ref[...] = dis * z
    sd_ref[...] = jnp.concatenate([sc, dis], axis=1)


def _tc_c(raw2_ref, u2_ref, sd_ref, wf_ref, b2_ref, bf_ref, out_ref):
    u2 = u2_ref[...]
    sd = sd_ref[...]
    dis = sd[:, 1:2]
    sc = sd[:, 0:1]
    rawsum = raw2_ref[0] + raw2_ref[1] + u2
    cv = lax.dot_general(b2_ref[...], wf_ref[...], (((1,), (1,)), ((), ())),
                         preferred_element_type=F32)
    out_ref[...] = dis * rawsum + sc * cv + bf_ref[...]


def kernel(x, edge_index, W1, b1, W2, b2, Wf, bf):
    n = x.shape[0]
    e = edge_index.shape[1]
    n_pad = _ceil_to(n + 1, max(BN, NS))
    ept = _ceil_to(-(-e // NW), CH)
    e_pad = ept * NW

    ei = edge_index.astype(jnp.int32)
    pad = jnp.full((e_pad - e,), n, dtype=jnp.int32)
    row2d = jnp.concatenate([ei[0], pad]).reshape(e_pad // SUB, SUB)
    col2d = jnp.concatenate([ei[1], pad]).reshape(e_pad // SUB, SUB)
    xpad = jnp.zeros((n_pad, 4), F32).at[:n].set(x)
    zeros1 = jnp.zeros((n_pad,), F32)
    zeros8 = jnp.zeros((n_pad, FW), F32)
    zeros2 = jnp.zeros((n_pad, FW2), F32)

    mesh = _sc_mesh()
    sc_params = pltpu.CompilerParams(use_tc_tiling_on_sc=False,
                                     skip_device_barrier=True)
    tc_params = pltpu.CompilerParams(skip_device_barrier=True)

    deg_call = pl.kernel(
        functools.partial(_deg_body, n_pad, ept),
        out_type=jax.ShapeDtypeStruct((NC, n_pad), F32),
        mesh=mesh,
        compiler_params=sc_params,
        scratch_types=[
            pltpu.VMEM_SHARED((n_pad,), F32),
            pltpu.VMEM((NSUB, SUB), jnp.int32),
            pltpu.VMEM((SUB,), F32),
            pltpu.SemaphoreType.DMA,
        ],
    )
    degp = deg_call(row2d, zeros1)

    def _make_prop(fw):
        return pl.kernel(
            functools.partial(_prop_body, n_pad, ept),
            out_type=jax.ShapeDtypeStruct((NC, n_pad, fw), F32),
            mesh=mesh,
            compiler_params=sc_params,
            scratch_types=[
                pltpu.VMEM_SHARED((n_pad, fw), F32),
                pltpu.VMEM((NSUB, SUB), jnp.int32),
                pltpu.VMEM((NSUB, SUB), jnp.int32),
                pltpu.VMEM((CH, fw), F32),
                pltpu.SemaphoreType.DMA,
                pltpu.SemaphoreType.DMA,
            ],
        )

    prop_call = _make_prop(FW)
    prop2_call = _make_prop(FW2)

    nb = n_pad // BN
    u = pl.pallas_call(
        _tc_a,
        grid=(nb,),
        in_specs=[
            pl.BlockSpec((NC, BN, 1), lambda i: (0, i, 0)),
            pl.BlockSpec((BN, 4), lambda i: (i, 0)),
        ],
        out_specs=pl.BlockSpec((BN, FW), lambda i: (i, 0)),
        out_shape=jax.ShapeDtypeStruct((n_pad, FW), F32),
        compiler_params=tc_params,
    )(degp.reshape(NC, n_pad, 1), xpad)

    raw1 = prop_call(u, row2d, col2d, zeros8)

    wspec = lambda shp: pl.BlockSpec(shp, lambda i: tuple(0 for _ in shp))
    u2, sd = pl.pallas_call(
        _tc_b,
        grid=(nb,),
        in_specs=[
            pl.BlockSpec((NC, BN, FW), lambda i: (0, i, 0)),
            pl.BlockSpec((BN, FW), lambda i: (i, 0)),
            wspec((128, 4)),
            wspec((1, 128)),
            wspec((128, 128)),
            wspec((2, 128)),
        ],
        out_specs=[
            pl.BlockSpec((BN, FW2), lambda i: (i, 0)),
            pl.BlockSpec((BN, 2), lambda i: (i, 0)),
        ],
        out_shape=[
            jax.ShapeDtypeStruct((n_pad, FW2), F32),
            jax.ShapeDtypeStruct((n_pad, 2), F32),
        ],
        compiler_params=tc_params,
    )(raw1, u, W1, b1.reshape(1, 128), W2, Wf)

    raw2 = prop2_call(u2, row2d, col2d, zeros2)

    outp = pl.pallas_call(
        _tc_c,
        grid=(nb,),
        in_specs=[
            pl.BlockSpec((NC, BN, FW2), lambda i: (0, i, 0)),
            pl.BlockSpec((BN, FW2), lambda i: (i, 0)),
            pl.BlockSpec((BN, 2), lambda i: (i, 0)),
            wspec((2, 128)),
            wspec((1, 128)),
            wspec((1, 2)),
        ],
        out_specs=pl.BlockSpec((BN, 2), lambda i: (i, 0)),
        out_shape=jax.ShapeDtypeStruct((n_pad, 2), F32),
        compiler_params=tc_params,
    )(raw2, u2, sd, Wf, b2.reshape(1, 128), bf.reshape(1, 2))

    return outp[:n]


# speedup vs baseline: 40.5657x; 1.0604x over previous
import functools

import jax
import jax.numpy as jnp
from jax import lax
from jax.experimental import pallas as pl
from jax.experimental.pallas import tpu as pltpu
from jax.experimental.pallas import tpu_sc as plsc

F32 = jnp.float32

NC = 2
NS = 16
NW = NC * NS
SUB = 128
NSUB = 8
CH = SUB * NSUB
FW = 8
FW2 = 8
BN = 4096


def _ceil_to(a, m):
    return (a + m - 1) // m * m


def _sc_mesh():
    return plsc.VectorSubcoreMesh(core_axis_name="c", subcore_axis_name="s")


def _deg_body(n_pad, ept, row2d, zeros1, out, acc, idx, ones_v, sem_s):
    c = lax.axis_index("c")
    s = lax.axis_index("s")
    wid = c * NS + s
    for i in range(SUB // 16):
        ones_v[pl.ds(i * 16, 16)] = jnp.ones((16,), F32)
    rpt = n_pad // NS
    pltpu.sync_copy(zeros1.at[pl.ds(s * rpt, rpt)], acc.at[pl.ds(s * rpt, rpt)])
    plsc.subcore_barrier()
    base = wid * (ept // SUB)

    @pl.loop(0, ept // CH)
    def _(k):
        pltpu.sync_copy(row2d.at[pl.ds(base + k * NSUB, NSUB)], idx)
        ds = [pltpu.async_copy(ones_v, acc.at[idx.at[j]], sem_s, add=True)
              for j in range(NSUB)]
        for d in ds:
            d.wait()

    plsc.subcore_barrier()
    pltpu.sync_copy(acc.at[pl.ds(s * rpt, rpt)], out.at[c, pl.ds(s * rpt, rpt)])


def _prop_body(n_pad, ept, u_hbm, row2d, col2d, zeros8, out,
               acc, idx_r, idx_c, gbuf, sem_g, sem_s):
    c = lax.axis_index("c")
    s = lax.axis_index("s")
    wid = c * NS + s
    rpt = n_pad // NS
    pltpu.sync_copy(zeros8.at[pl.ds(s * rpt, rpt)], acc.at[pl.ds(s * rpt, rpt)])
    plsc.subcore_barrier()
    base = wid * (ept // SUB)

    @pl.loop(0, ept // CH)
    def _(k):
        off = base + k * NSUB
        pltpu.sync_copy(row2d.at[pl.ds(off, NSUB)], idx_r)
        pltpu.sync_copy(col2d.at[pl.ds(off, NSUB)], idx_c)
        gd = [pltpu.async_copy(u_hbm.at[idx_r.at[j]],
                               gbuf.at[pl.ds(j * SUB, SUB)], sem_g)
              for j in range(NSUB)]
        for d in gd:
            d.wait()
        sd = [pltpu.async_copy(gbuf.at[pl.ds(j * SUB, SUB)],
                               acc.at[idx_c.at[j]], sem_s, add=True)
              for j in range(NSUB)]
        for d in sd:
            d.wait()

    plsc.subcore_barrier()
    pltpu.sync_copy(acc.at[pl.ds(s * rpt, rpt)], out.at[c, pl.ds(s * rpt, rpt)])


def _tc_a(degp_ref, x_ref, u_ref):
    deg = degp_ref[0] + degp_ref[1] + 1.0
    dis = lax.rsqrt(deg)
    xb = x_ref[...]
    ones = jnp.ones((xb.shape[0], 1), F32)
    zeros = jnp.zeros((xb.shape[0], FW - 5), F32)
    u_ref[...] = dis * jnp.concatenate([xb, ones, zeros], axis=1)


def _tc_b(raw1_ref, u_ref, w1_ref, b1_ref, w2_ref, wf_ref, u2_ref):
    u = u_ref[...]
    rawsum = raw1_ref[0] + raw1_ref[1] + u
    dis = u[:, 4:5]
    out1 = dis * rawsum
    sc = out1[:, 4:5]
    pre = lax.dot_general(out1[:, :4], w1_ref[...], (((1,), (1,)), ((), ())),
                          preferred_element_type=F32) + sc * b1_ref[...]
    h = jnp.maximum(pre, 0.0)
    t = lax.dot_general(h, w2_ref[...], (((1,), (1,)), ((), ())),
                        preferred_element_type=F32)
    z = lax.dot_general(t, wf_ref[...], (((1,), (1,)), ((), ())),
                        preferred_element_type=F32)
    zeros = jnp.zeros((z.shape[0], FW2 - 4), F32)
    u2_ref[...] = jnp.concatenate([dis * z, sc, dis, zeros], axis=1)


def _tc_c(raw2_ref, u2_ref, wf_ref, b2_ref, bf_ref, out_ref):
    u2 = u2_ref[...]
    dis = u2[:, 3:4]
    sc = u2[:, 2:3]
    rawsum = raw2_ref[0][:, :2] + raw2_ref[1][:, :2] + u2[:, :2]
    cv = lax.dot_general(b2_ref[...], wf_ref[...], (((1,), (1,)), ((), ())),
                         preferred_element_type=F32)
    out_ref[...] = dis * rawsum + sc * cv + bf_ref[...]


def kernel(x, edge_index, W1, b1, W2, b2, Wf, bf):
    n = x.shape[0]
    e = edge_index.shape[1]
    n_pad = _ceil_to(n + 1, max(BN, NS))
    ept = _ceil_to(-(-e // NW), CH)
    e_pad = ept * NW

    ei = edge_index.astype(jnp.int32)
    pad = jnp.full((e_pad - e,), n, dtype=jnp.int32)
    row2d = jnp.concatenate([ei[0], pad]).reshape(e_pad // SUB, SUB)
    col2d = jnp.concatenate([ei[1], pad]).reshape(e_pad // SUB, SUB)
    xpad = jnp.zeros((n_pad, 4), F32).at[:n].set(x)
    zeros1 = jnp.zeros((n_pad,), F32)
    zeros8 = jnp.zeros((n_pad, FW), F32)
    zeros2 = jnp.zeros((n_pad, FW2), F32)

    mesh = _sc_mesh()
    sc_params = pltpu.CompilerParams(use_tc_tiling_on_sc=False,
                                     skip_device_barrier=True)
    tc_params = pltpu.CompilerParams(skip_device_barrier=True)

    deg_call = pl.kernel(
        functools.partial(_deg_body, n_pad, ept),
        out_type=jax.ShapeDtypeStruct((NC, n_pad), F32),
        mesh=mesh,
        compiler_params=sc_params,
        scratch_types=[
            pltpu.VMEM_SHARED((n_pad,), F32),
            pltpu.VMEM((NSUB, SUB), jnp.int32),
            pltpu.VMEM((SUB,), F32),
            pltpu.SemaphoreType.DMA,
        ],
    )
    degp = deg_call(row2d, zeros1)

    def _make_prop(fw):
        return pl.kernel(
            functools.partial(_prop_body, n_pad, ept),
            out_type=jax.ShapeDtypeStruct((NC, n_pad, fw), F32),
            mesh=mesh,
            compiler_params=sc_params,
            scratch_types=[
                pltpu.VMEM_SHARED((n_pad, fw), F32),
                pltpu.VMEM((NSUB, SUB), jnp.int32),
                pltpu.VMEM((NSUB, SUB), jnp.int32),
                pltpu.VMEM((CH, fw), F32),
                pltpu.SemaphoreType.DMA,
                pltpu.SemaphoreType.DMA,
            ],
        )

    prop_call = _make_prop(FW)
    prop2_call = _make_prop(FW2)

    nb = n_pad // BN
    u = pl.pallas_call(
        _tc_a,
        grid=(nb,),
        in_specs=[
            pl.BlockSpec((NC, BN, 1), lambda i: (0, i, 0)),
            pl.BlockSpec((BN, 4), lambda i: (i, 0)),
        ],
        out_specs=pl.BlockSpec((BN, FW), lambda i: (i, 0)),
        out_shape=jax.ShapeDtypeStruct((n_pad, FW), F32),
        compiler_params=tc_params,
    )(degp.reshape(NC, n_pad, 1), xpad)

    raw1 = prop_call(u, row2d, col2d, zeros8)

    wspec = lambda shp: pl.BlockSpec(shp, lambda i: tuple(0 for _ in shp))
    u2 = pl.pallas_call(
        _tc_b,
        grid=(nb,),
        in_specs=[
            pl.BlockSpec((NC, BN, FW), lambda i: (0, i, 0)),
            pl.BlockSpec((BN, FW), lambda i: (i, 0)),
            wspec((128, 4)),
            wspec((1, 128)),
            wspec((128, 128)),
            wspec((2, 128)),
        ],
        out_specs=pl.BlockSpec((BN, FW2), lambda i: (i, 0)),
        out_shape=jax.ShapeDtypeStruct((n_pad, FW2), F32),
        compiler_params=tc_params,
    )(raw1, u, W1, b1.reshape(1, 128), W2, Wf)

    raw2 = prop2_call(u2, row2d, col2d, zeros2)

    outp = pl.pallas_call(
        _tc_c,
        grid=(nb,),
        in_specs=[
            pl.BlockSpec((NC, BN, FW2), lambda i: (0, i, 0)),
            pl.BlockSpec((BN, FW2), lambda i: (i, 0)),
            wspec((2, 128)),
            wspec((1, 128)),
            wspec((1, 2)),
        ],
        out_specs=pl.BlockSpec((BN, 2), lambda i: (i, 0)),
        out_shape=jax.ShapeDtypeStruct((n_pad, 2), F32),
        compiler_params=tc_params,
    )(raw2, u2, Wf, b2.reshape(1, 128), bf.reshape(1, 2))

    return outp[:n]


# speedup vs baseline: 44.7994x; 1.1044x over previous
import functools

import jax
import jax.numpy as jnp
from jax import lax
from jax.experimental import pallas as pl
from jax.experimental.pallas import tpu as pltpu
from jax.experimental.pallas import tpu_sc as plsc

F32 = jnp.float32

NC = 2
NS = 16
NW = NC * NS
SUB = 128
NSUB = 8
CH = SUB * NSUB
FW = 8
FW2 = 8
PCHUNKS = 4
BN = 4096


def _ceil_to(a, m):
    return (a + m - 1) // m * m


def _sc_mesh():
    return plsc.VectorSubcoreMesh(core_axis_name="c", subcore_axis_name="s")


def _deg_body(n_pad, ept, row2d, zeros1, out, acc, idx, ones_v, sem_s):
    c = lax.axis_index("c")
    s = lax.axis_index("s")
    wid = c * NS + s
    for i in range(SUB // 16):
        ones_v[pl.ds(i * 16, 16)] = jnp.ones((16,), F32)
    rpt = n_pad // NS
    pltpu.sync_copy(zeros1.at[pl.ds(s * rpt, rpt)], acc.at[pl.ds(s * rpt, rpt)])
    plsc.subcore_barrier()
    base = wid * (ept // SUB)

    @pl.loop(0, ept // CH)
    def _(k):
        pltpu.sync_copy(row2d.at[pl.ds(base + k * NSUB, NSUB)], idx)
        ds = [pltpu.async_copy(ones_v, acc.at[idx.at[j]], sem_s, add=True)
              for j in range(NSUB)]
        for d in ds:
            d.wait()

    plsc.subcore_barrier()
    pltpu.sync_copy(acc.at[pl.ds(s * rpt, rpt)], out.at[c, pl.ds(s * rpt, rpt)])


def _rsqrt16(d):
    di = plsc.bitcast(d, jnp.int32)
    y = plsc.bitcast(jnp.int32(0x5F3759DF) - lax.shift_right_arithmetic(di, 1),
                     F32)
    for _ in range(3):
        y = y * (1.5 - 0.5 * d * y * y)
    return y


def _prop1_body(n_pad, ept, degp, xflat, row2d, col2d, zeros8, u0, u1, out,
                acc, idx_r, idx_c, gbuf, dbuf, xbuf, ubuf, disbuf,
                sem_g, sem_s):
    c = lax.axis_index("c")
    s = lax.axis_index("s")
    wid = c * NS + s
    rpt = n_pad // NS
    r0 = s * rpt
    pltpu.sync_copy(zeros8.at[pl.ds(r0, rpt)], acc.at[pl.ds(r0, rpt)])
    iota = lax.iota(jnp.int32, 16)
    lane8 = iota // 8
    word = iota % 8
    maskx = word < 4
    maskd = word == 4
    zero16 = jnp.zeros((16,), F32)
    wordc = jnp.minimum(word, 3)
    cpn = rpt // PCHUNKS

    def _prologue(u_hbm):
        @pl.loop(0, PCHUNKS)
        def _(p):
            rp = r0 + p * cpn
            pltpu.sync_copy(degp.at[0, pl.ds(rp, cpn)], dbuf.at[0])
            pltpu.sync_copy(degp.at[1, pl.ds(rp, cpn)], dbuf.at[1])
            pltpu.sync_copy(xflat.at[pl.ds(rp * 4, cpn * 4)], xbuf)

            @pl.loop(0, cpn // 16)
            def _(g):
                d = dbuf[0, pl.ds(g * 16, 16)] + dbuf[1, pl.ds(g * 16, 16)] + 1.0
                disbuf[pl.ds(g * 16, 16)] = _rsqrt16(d)

            @pl.loop(0, cpn // 2)
            def _(m):
                node = 2 * m + lane8
                dg = plsc.load_gather(disbuf, [node])
                xg = plsc.load_gather(xbuf, [node * 4 + wordc])
                uv = jnp.where(maskx, xg * dg, jnp.where(maskd, dg, zero16))
                plsc.store_scatter(ubuf, [node, word], uv)

            pltpu.sync_copy(ubuf, u_hbm.at[pl.ds(rp, cpn)])

    base = wid * (ept // SUB)

    def _edge_loop(u_hbm):
        _prologue(u_hbm)
        plsc.subcore_barrier()

        @pl.loop(0, ept // CH)
        def _(k):
            off = base + k * NSUB
            pltpu.sync_copy(row2d.at[pl.ds(off, NSUB)], idx_r)
            pltpu.sync_copy(col2d.at[pl.ds(off, NSUB)], idx_c)
            gd = [pltpu.async_copy(u_hbm.at[idx_r.at[j]],
                                   gbuf.at[pl.ds(j * SUB, SUB)], sem_g)
                  for j in range(NSUB)]
            for d in gd:
                d.wait()
            sd = [pltpu.async_copy(gbuf.at[pl.ds(j * SUB, SUB)],
                                   acc.at[idx_c.at[j]], sem_s, add=True)
                  for j in range(NSUB)]
            for d in sd:
                d.wait()

    @pl.when(c == 0)
    def _():
        _edge_loop(u0)

    @pl.when(c == 1)
    def _():
        _edge_loop(u1)

    plsc.subcore_barrier()
    pltpu.sync_copy(acc.at[pl.ds(r0, rpt)], out.at[c, pl.ds(r0, rpt)])


def _prop_body(n_pad, ept, u_hbm, row2d, col2d, zeros8, out,
               acc, idx_r, idx_c, gbuf, sem_g, sem_s):
    c = lax.axis_index("c")
    s = lax.axis_index("s")
    wid = c * NS + s
    rpt = n_pad // NS
    pltpu.sync_copy(zeros8.at[pl.ds(s * rpt, rpt)], acc.at[pl.ds(s * rpt, rpt)])
    plsc.subcore_barrier()
    base = wid * (ept // SUB)

    @pl.loop(0, ept // CH)
    def _(k):
        off = base + k * NSUB
        pltpu.sync_copy(row2d.at[pl.ds(off, NSUB)], idx_r)
        pltpu.sync_copy(col2d.at[pl.ds(off, NSUB)], idx_c)
        gd = [pltpu.async_copy(u_hbm.at[idx_r.at[j]],
                               gbuf.at[pl.ds(j * SUB, SUB)], sem_g)
              for j in range(NSUB)]
        for d in gd:
            d.wait()
        sd = [pltpu.async_copy(gbuf.at[pl.ds(j * SUB, SUB)],
                               acc.at[idx_c.at[j]], sem_s, add=True)
              for j in range(NSUB)]
        for d in sd:
            d.wait()

    plsc.subcore_barrier()
    pltpu.sync_copy(acc.at[pl.ds(s * rpt, rpt)], out.at[c, pl.ds(s * rpt, rpt)])


def _tc_a(degp_ref, x_ref, u_ref):
    deg = degp_ref[0] + degp_ref[1] + 1.0
    dis = lax.rsqrt(deg)
    xb = x_ref[...]
    ones = jnp.ones((xb.shape[0], 1), F32)
    zeros = jnp.zeros((xb.shape[0], FW - 5), F32)
    u_ref[...] = dis * jnp.concatenate([xb, ones, zeros], axis=1)


def _tc_b(raw1_ref, u_ref, w1_ref, b1_ref, w2_ref, wf_ref, u2_ref):
    u = u_ref[...]
    rawsum = raw1_ref[0] + raw1_ref[1] + u
    dis = u[:, 4:5]
    out1 = dis * rawsum
    sc = out1[:, 4:5]
    pre = lax.dot_general(out1[:, :4], w1_ref[...], (((1,), (1,)), ((), ())),
                          preferred_element_type=F32) + sc * b1_ref[...]
    h = jnp.maximum(pre, 0.0)
    t = lax.dot_general(h, w2_ref[...], (((1,), (1,)), ((), ())),
                        preferred_element_type=F32)
    z = lax.dot_general(t, wf_ref[...], (((1,), (1,)), ((), ())),
                        preferred_element_type=F32)
    zeros = jnp.zeros((z.shape[0], FW2 - 4), F32)
    u2_ref[...] = jnp.concatenate([dis * z, sc, dis, zeros], axis=1)


def _tc_c(raw2_ref, u2_ref, wf_ref, b2_ref, bf_ref, out_ref):
    u2 = u2_ref[...]
    dis = u2[:, 3:4]
    sc = u2[:, 2:3]
    rawsum = raw2_ref[0][:, :2] + raw2_ref[1][:, :2] + u2[:, :2]
    cv = lax.dot_general(b2_ref[...], wf_ref[...], (((1,), (1,)), ((), ())),
                         preferred_element_type=F32)
    out_ref[...] = dis * rawsum + sc * cv + bf_ref[...]


def kernel(x, edge_index, W1, b1, W2, b2, Wf, bf):
    n = x.shape[0]
    e = edge_index.shape[1]
    n_pad = _ceil_to(n + 1, max(BN, NS))
    ept = _ceil_to(-(-e // NW), CH)
    e_pad = ept * NW

    ei = edge_index.astype(jnp.int32)
    pad = jnp.full((e_pad - e,), n, dtype=jnp.int32)
    row2d = jnp.concatenate([ei[0], pad]).reshape(e_pad // SUB, SUB)
    col2d = jnp.concatenate([ei[1], pad]).reshape(e_pad // SUB, SUB)
    xpad = jnp.zeros((n_pad, 4), F32).at[:n].set(x)
    zeros1 = jnp.zeros((n_pad,), F32)
    zeros8 = jnp.zeros((n_pad, FW), F32)
    zeros2 = jnp.zeros((n_pad, FW2), F32)

    mesh = _sc_mesh()
    sc_params = pltpu.CompilerParams(use_tc_tiling_on_sc=False,
                                     skip_device_barrier=True)
    sc_params_nl = pltpu.CompilerParams(use_tc_tiling_on_sc=False,
                                        skip_device_barrier=True,
                                        needs_layout_passes=False)
    tc_params = pltpu.CompilerParams(skip_device_barrier=True)

    deg_call = pl.kernel(
        functools.partial(_deg_body, n_pad, ept),
        out_type=jax.ShapeDtypeStruct((NC, n_pad), F32),
        mesh=mesh,
        compiler_params=sc_params,
        scratch_types=[
            pltpu.VMEM_SHARED((n_pad,), F32),
            pltpu.VMEM((NSUB, SUB), jnp.int32),
            pltpu.VMEM((SUB,), F32),
            pltpu.SemaphoreType.DMA,
        ],
    )
    degp = deg_call(row2d, zeros1)

    def _make_prop(fw):
        return pl.kernel(
            functools.partial(_prop_body, n_pad, ept),
            out_type=jax.ShapeDtypeStruct((NC, n_pad, fw), F32),
            mesh=mesh,
            compiler_params=sc_params,
            scratch_types=[
                pltpu.VMEM_SHARED((n_pad, fw), F32),
                pltpu.VMEM((NSUB, SUB), jnp.int32),
                pltpu.VMEM((NSUB, SUB), jnp.int32),
                pltpu.VMEM((CH, fw), F32),
                pltpu.SemaphoreType.DMA,
                pltpu.SemaphoreType.DMA,
            ],
        )

    prop2_call = _make_prop(FW2)

    rpt = n_pad // NS
    prop1_call = pl.kernel(
        functools.partial(_prop1_body, n_pad, ept),
        out_type=[
            jax.ShapeDtypeStruct((n_pad, FW), F32),
            jax.ShapeDtypeStruct((n_pad, FW), F32),
            jax.ShapeDtypeStruct((NC, n_pad, FW), F32),
        ],
        mesh=mesh,
        compiler_params=sc_params_nl,
        scratch_types=[
            pltpu.VMEM_SHARED((n_pad, FW), F32),
            pltpu.VMEM((NSUB, SUB), jnp.int32),
            pltpu.VMEM((NSUB, SUB), jnp.int32),
            pltpu.VMEM((CH, FW), F32),
            pltpu.VMEM((2, rpt // PCHUNKS), F32),
            pltpu.VMEM((rpt // PCHUNKS * 4,), F32),
            pltpu.VMEM((rpt // PCHUNKS, FW), F32),
            pltpu.VMEM((rpt // PCHUNKS,), F32),
            pltpu.SemaphoreType.DMA,
            pltpu.SemaphoreType.DMA,
        ],
    )

    nb = n_pad // BN
    u, _u1, raw1 = prop1_call(degp, xpad.reshape(n_pad * 4), row2d, col2d,
                              zeros8)

    wspec = lambda shp: pl.BlockSpec(shp, lambda i: tuple(0 for _ in shp))
    u2 = pl.pallas_call(
        _tc_b,
        grid=(nb,),
        in_specs=[
            pl.BlockSpec((NC, BN, FW), lambda i: (0, i, 0)),
            pl.BlockSpec((BN, FW), lambda i: (i, 0)),
            wspec((128, 4)),
            wspec((1, 128)),
            wspec((128, 128)),
            wspec((2, 128)),
        ],
        out_specs=pl.BlockSpec((BN, FW2), lambda i: (i, 0)),
        out_shape=jax.ShapeDtypeStruct((n_pad, FW2), F32),
        compiler_params=tc_params,
    )(raw1, u, W1, b1.reshape(1, 128), W2, Wf)

    raw2 = prop2_call(u2, row2d, col2d, zeros2)

    outp = pl.pallas_call(
        _tc_c,
        grid=(nb,),
        in_specs=[
            pl.BlockSpec((NC, BN, FW2), lambda i: (0, i, 0)),
            pl.BlockSpec((BN, FW2), lambda i: (i, 0)),
            wspec((2, 128)),
            wspec((1, 128)),
            wspec((1, 2)),
        ],
        out_specs=pl.BlockSpec((BN, 2), lambda i: (i, 0)),
        out_shape=jax.ShapeDtypeStruct((n_pad, 2), F32),
        compiler_params=tc_params,
    )(raw2, u2, Wf, b2.reshape(1, 128), bf.reshape(1, 2))

    return outp[:n]


# speedup vs baseline: 45.1603x; 1.0081x over previous
import functools

import jax
import jax.numpy as jnp
from jax import lax
from jax.experimental import pallas as pl
from jax.experimental.pallas import tpu as pltpu
from jax.experimental.pallas import tpu_sc as plsc

F32 = jnp.float32

NC = 2
NS = 16
NW = NC * NS
SUB = 128
NSUB = 8
CH = SUB * NSUB
FW = 8
FW2 = 8
PCHUNKS = 4
BN = 4096


def _ceil_to(a, m):
    return (a + m - 1) // m * m


def _sc_mesh():
    return plsc.VectorSubcoreMesh(core_axis_name="c", subcore_axis_name="s")


def _deg_body(n_pad, ept, row2d, zeros1, out, acc, idx, ones_v, sem_s):
    c = lax.axis_index("c")
    s = lax.axis_index("s")
    wid = c * NS + s
    for i in range(SUB // 16):
        ones_v[pl.ds(i * 16, 16)] = jnp.ones((16,), F32)
    rpt = n_pad // NS
    pltpu.sync_copy(zeros1.at[pl.ds(s * rpt, rpt)], acc.at[pl.ds(s * rpt, rpt)])
    plsc.subcore_barrier()
    base = wid * (ept // SUB)

    @pl.loop(0, ept // CH)
    def _(k):
        pltpu.sync_copy(row2d.at[pl.ds(base + k * NSUB, NSUB)], idx)
        ds = [pltpu.async_copy(ones_v, acc.at[idx.at[j]], sem_s, add=True)
              for j in range(NSUB)]
        for d in ds:
            d.wait()

    plsc.subcore_barrier()
    pltpu.sync_copy(acc.at[pl.ds(s * rpt, rpt)], out.at[c, pl.ds(s * rpt, rpt)])


def _rsqrt16(d):
    di = plsc.bitcast(d, jnp.int32)
    y = plsc.bitcast(jnp.int32(0x5F3759DF) - lax.shift_right_arithmetic(di, 1),
                     F32)
    for _ in range(3):
        y = y * (1.5 - 0.5 * d * y * y)
    return y


def _prop1_body(n_pad, ept, degp, xflat, row2d, col2d, zeros8, u0, u1, out,
                acc, idx_r, idx_c, gbuf, dbuf, xbuf, ubuf, disbuf,
                sem_g, sem_s):
    c = lax.axis_index("c")
    s = lax.axis_index("s")
    wid = c * NS + s
    rpt = n_pad // NS
    r0 = s * rpt
    pltpu.sync_copy(zeros8.at[pl.ds(r0, rpt)], acc.at[pl.ds(r0, rpt)])
    iota = lax.iota(jnp.int32, 16)
    lane8 = iota // 8
    word = iota % 8
    maskx = word < 4
    maskd = word == 4
    zero16 = jnp.zeros((16,), F32)
    wordc = jnp.minimum(word, 3)
    cpn = rpt // PCHUNKS

    def _prologue(u_hbm):
        @pl.loop(0, PCHUNKS)
        def _(p):
            rp = r0 + p * cpn
            pltpu.sync_copy(degp.at[0, pl.ds(rp, cpn)], dbuf.at[0])
            pltpu.sync_copy(degp.at[1, pl.ds(rp, cpn)], dbuf.at[1])
            pltpu.sync_copy(xflat.at[pl.ds(rp * 4, cpn * 4)], xbuf)

            @pl.loop(0, cpn // 16)
            def _(g):
                d = dbuf[0, pl.ds(g * 16, 16)] + dbuf[1, pl.ds(g * 16, 16)] + 1.0
                disbuf[pl.ds(g * 16, 16)] = _rsqrt16(d)

            @pl.loop(0, cpn // 2)
            def _(m):
                node = 2 * m + lane8
                dg = plsc.load_gather(disbuf, [node])
                xg = plsc.load_gather(xbuf, [node * 4 + wordc])
                uv = jnp.where(maskx, xg * dg, jnp.where(maskd, dg, zero16))
                plsc.store_scatter(ubuf, [node, word], uv)

            pltpu.sync_copy(ubuf, u_hbm.at[pl.ds(rp, cpn)])

    base = wid * (ept // SUB)

    def _edge_loop(u_hbm):
        _prologue(u_hbm)
        plsc.subcore_barrier()

        @pl.loop(0, ept // CH)
        def _(k):
            off = base + k * NSUB
            pltpu.sync_copy(row2d.at[pl.ds(off, NSUB)], idx_r)
            pltpu.sync_copy(col2d.at[pl.ds(off, NSUB)], idx_c)
            gd = [pltpu.async_copy(u_hbm.at[idx_r.at[j]],
                                   gbuf.at[pl.ds(j * SUB, SUB)], sem_g)
                  for j in range(NSUB)]
            for d in gd:
                d.wait()
            sd = [pltpu.async_copy(gbuf.at[pl.ds(j * SUB, SUB)],
                                   acc.at[idx_c.at[j]], sem_s, add=True)
                  for j in range(NSUB)]
            for d in sd:
                d.wait()

    @pl.when(c == 0)
    def _():
        _edge_loop(u0)

    @pl.when(c == 1)
    def _():
        _edge_loop(u1)

    plsc.subcore_barrier()
    pltpu.sync_copy(acc.at[pl.ds(r0, rpt)], out.at[c, pl.ds(r0, rpt)])


def _prop_body(n_pad, ept, u_hbm, row2d, col2d, zeros8, out,
               acc, idx_r, idx_c, gbuf, sem_g, sem_s):
    c = lax.axis_index("c")
    s = lax.axis_index("s")
    wid = c * NS + s
    rpt = n_pad // NS
    pltpu.sync_copy(zeros8.at[pl.ds(s * rpt, rpt)], acc.at[pl.ds(s * rpt, rpt)])
    plsc.subcore_barrier()
    base = wid * (ept // SUB)

    @pl.loop(0, ept // CH)
    def _(k):
        off = base + k * NSUB
        pltpu.sync_copy(row2d.at[pl.ds(off, NSUB)], idx_r)
        pltpu.sync_copy(col2d.at[pl.ds(off, NSUB)], idx_c)
        gd = [pltpu.async_copy(u_hbm.at[idx_r.at[j]],
                               gbuf.at[pl.ds(j * SUB, SUB)], sem_g)
              for j in range(NSUB)]
        for d in gd:
            d.wait()
        sd = [pltpu.async_copy(gbuf.at[pl.ds(j * SUB, SUB)],
                               acc.at[idx_c.at[j]], sem_s, add=True)
              for j in range(NSUB)]
        for d in sd:
            d.wait()

    plsc.subcore_barrier()
    pltpu.sync_copy(acc.at[pl.ds(s * rpt, rpt)], out.at[c, pl.ds(s * rpt, rpt)])


def _tc_a(degp_ref, x_ref, u_ref):
    deg = degp_ref[0] + degp_ref[1] + 1.0
    dis = lax.rsqrt(deg)
    xb = x_ref[...]
    ones = jnp.ones((xb.shape[0], 1), F32)
    zeros = jnp.zeros((xb.shape[0], FW - 5), F32)
    u_ref[...] = dis * jnp.concatenate([xb, ones, zeros], axis=1)


def _tc_b(raw1_ref, u_ref, w1aug_ref, w2_ref, wf_ref, u2_ref):
    u = u_ref[...]
    rawsum = raw1_ref[0] + raw1_ref[1] + u
    dis = u[:, 4:5]
    sc = dis * rawsum[:, 4:5]
    pre = dis * lax.dot_general(rawsum, w1aug_ref[...], (((1,), (0,)), ((), ())),
                                preferred_element_type=F32)
    h = jnp.maximum(pre, 0.0)
    t = lax.dot_general(h, w2_ref[...], (((1,), (1,)), ((), ())),
                        preferred_element_type=F32)
    z = lax.dot_general(t, wf_ref[...], (((1,), (1,)), ((), ())),
                        preferred_element_type=F32)
    zeros = jnp.zeros((z.shape[0], FW2 - 4), F32)
    u2_ref[...] = jnp.concatenate([dis * z, sc, dis, zeros], axis=1)


def _tc_c(raw2_ref, u2_ref, wf_ref, b2_ref, bf_ref, out_ref):
    u2 = u2_ref[...]
    dis = u2[:, 3:4]
    sc = u2[:, 2:3]
    rawsum = raw2_ref[0][:, :2] + raw2_ref[1][:, :2] + u2[:, :2]
    cv = lax.dot_general(b2_ref[...], wf_ref[...], (((1,), (1,)), ((), ())),
                         preferred_element_type=F32)
    out_ref[...] = dis * rawsum + sc * cv + bf_ref[...]


def kernel(x, edge_index, W1, b1, W2, b2, Wf, bf):
    n = x.shape[0]
    e = edge_index.shape[1]
    n_pad = _ceil_to(n + 1, max(BN, NS))
    ept = _ceil_to(-(-e // NW), CH)
    e_pad = ept * NW

    ei = edge_index.astype(jnp.int32)
    pad = jnp.full((e_pad - e,), n, dtype=jnp.int32)
    row2d = jnp.concatenate([ei[0], pad]).reshape(e_pad // SUB, SUB)
    col2d = jnp.concatenate([ei[1], pad]).reshape(e_pad // SUB, SUB)
    xpad = jnp.zeros((n_pad, 4), F32).at[:n].set(x)
    zeros1 = jnp.zeros((n_pad,), F32)
    zeros8 = jnp.zeros((n_pad, FW), F32)
    zeros2 = jnp.zeros((n_pad, FW2), F32)
    w1aug = jnp.concatenate([W1.T, b1[None, :], jnp.zeros((FW - 5, 128), F32)],
                            axis=0)

    mesh = _sc_mesh()
    sc_params = pltpu.CompilerParams(use_tc_tiling_on_sc=False,
                                     skip_device_barrier=True)
    sc_params_nl = pltpu.CompilerParams(use_tc_tiling_on_sc=False,
                                        skip_device_barrier=True,
                                        needs_layout_passes=False)
    tc_params = pltpu.CompilerParams(skip_device_barrier=True)

    deg_call = pl.kernel(
        functools.partial(_deg_body, n_pad, ept),
        out_type=jax.ShapeDtypeStruct((NC, n_pad), F32),
        mesh=mesh,
        compiler_params=sc_params,
        scratch_types=[
            pltpu.VMEM_SHARED((n_pad,), F32),
            pltpu.VMEM((NSUB, SUB), jnp.int32),
            pltpu.VMEM((SUB,), F32),
            pltpu.SemaphoreType.DMA,
        ],
    )
    degp = deg_call(row2d, zeros1)

    def _make_prop(fw):
        return pl.kernel(
            functools.partial(_prop_body, n_pad, ept),
            out_type=jax.ShapeDtypeStruct((NC, n_pad, fw), F32),
            mesh=mesh,
            compiler_params=sc_params,
            scratch_types=[
                pltpu.VMEM_SHARED((n_pad, fw), F32),
                pltpu.VMEM((NSUB, SUB), jnp.int32),
                pltpu.VMEM((NSUB, SUB), jnp.int32),
                pltpu.VMEM((CH, fw), F32),
                pltpu.SemaphoreType.DMA,
                pltpu.SemaphoreType.DMA,
            ],
        )

    prop2_call = _make_prop(FW2)

    rpt = n_pad // NS
    prop1_call = pl.kernel(
        functools.partial(_prop1_body, n_pad, ept),
        out_type=[
            jax.ShapeDtypeStruct((n_pad, FW), F32),
            jax.ShapeDtypeStruct((n_pad, FW), F32),
            jax.ShapeDtypeStruct((NC, n_pad, FW), F32),
        ],
        mesh=mesh,
        compiler_params=sc_params_nl,
        scratch_types=[
            pltpu.VMEM_SHARED((n_pad, FW), F32),
            pltpu.VMEM((NSUB, SUB), jnp.int32),
            pltpu.VMEM((NSUB, SUB), jnp.int32),
            pltpu.VMEM((CH, FW), F32),
            pltpu.VMEM((2, rpt // PCHUNKS), F32),
            pltpu.VMEM((rpt // PCHUNKS * 4,), F32),
            pltpu.VMEM((rpt // PCHUNKS, FW), F32),
            pltpu.VMEM((rpt // PCHUNKS,), F32),
            pltpu.SemaphoreType.DMA,
            pltpu.SemaphoreType.DMA,
        ],
    )

    nb = n_pad // BN
    u, _u1, raw1 = prop1_call(degp, xpad.reshape(n_pad * 4), row2d, col2d,
                              zeros8)

    wspec = lambda shp: pl.BlockSpec(shp, lambda i: tuple(0 for _ in shp))
    u2 = pl.pallas_call(
        _tc_b,
        grid=(nb,),
        in_specs=[
            pl.BlockSpec((NC, BN, FW), lambda i: (0, i, 0)),
            pl.BlockSpec((BN, FW), lambda i: (i, 0)),
            wspec((FW, 128)),
            wspec((128, 128)),
            wspec((2, 128)),
        ],
        out_specs=pl.BlockSpec((BN, FW2), lambda i: (i, 0)),
        out_shape=jax.ShapeDtypeStruct((n_pad, FW2), F32),
        compiler_params=tc_params,
    )(raw1, u, w1aug, W2, Wf)

    raw2 = prop2_call(u2, row2d, col2d, zeros2)

    outp = pl.pallas_call(
        _tc_c,
        grid=(nb,),
        in_specs=[
            pl.BlockSpec((NC, BN, FW2), lambda i: (0, i, 0)),
            pl.BlockSpec((BN, FW2), lambda i: (i, 0)),
            wspec((2, 128)),
            wspec((1, 128)),
            wspec((1, 2)),
        ],
        out_specs=pl.BlockSpec((BN, 2), lambda i: (i, 0)),
        out_shape=jax.ShapeDtypeStruct((n_pad, 2), F32),
        compiler_params=tc_params,
    )(raw2, u2, Wf, b2.reshape(1, 128), bf.reshape(1, 2))

    return outp[:n]


# speedup vs baseline: 46.7682x; 1.0356x over previous
import functools

import jax
import jax.numpy as jnp
from jax import lax
from jax.experimental import pallas as pl
from jax.experimental.pallas import tpu as pltpu
from jax.experimental.pallas import tpu_sc as plsc

F32 = jnp.float32

NC = 2
NS = 16
NW = NC * NS
SUB = 128
NSUB = 8
CH = SUB * NSUB
FW = 8
FW2 = 8
PCHUNKS = 4
BN = 4096


def _ceil_to(a, m):
    return (a + m - 1) // m * m


def _sc_mesh():
    return plsc.VectorSubcoreMesh(core_axis_name="c", subcore_axis_name="s")


def _deg_body(n_pad, ept, row2d, zeros1, out, acc, idx, ones_v, sem_s):
    c = lax.axis_index("c")
    s = lax.axis_index("s")
    wid = c * NS + s
    for i in range(SUB // 16):
        ones_v[pl.ds(i * 16, 16)] = jnp.ones((16,), F32)
    rpt = n_pad // NS
    pltpu.sync_copy(zeros1.at[pl.ds(s * rpt, rpt)], acc.at[pl.ds(s * rpt, rpt)])
    plsc.subcore_barrier()
    base = wid * (ept // SUB)

    @pl.loop(0, ept // CH)
    def _(k):
        pltpu.sync_copy(row2d.at[pl.ds(base + k * NSUB, NSUB)], idx)
        ds = [pltpu.async_copy(ones_v, acc.at[idx.at[j]], sem_s, add=True)
              for j in range(NSUB)]
        for d in ds:
            d.wait()

    plsc.subcore_barrier()
    pltpu.sync_copy(acc.at[pl.ds(s * rpt, rpt)], out.at[c, pl.ds(s * rpt, rpt)])


def _rsqrt16(d):
    di = plsc.bitcast(d, jnp.int32)
    y = plsc.bitcast(jnp.int32(0x5F3759DF) - lax.shift_right_arithmetic(di, 1),
                     F32)
    for _ in range(3):
        y = y * (1.5 - 0.5 * d * y * y)
    return y


def _prop1_body(n_pad, ept, degp, xflat, row2d, col2d, zeros8, u0, u1, out,
                acc, idx_r, idx_c, gbuf, dbuf, xbuf, ubuf, disbuf,
                sem_g, sem_s):
    c = lax.axis_index("c")
    s = lax.axis_index("s")
    wid = c * NS + s
    rpt = n_pad // NS
    r0 = s * rpt
    pltpu.sync_copy(zeros8.at[pl.ds(r0, rpt)], acc.at[pl.ds(r0, rpt)])
    iota = lax.iota(jnp.int32, 16)
    lane8 = iota // 8
    word = iota % 8
    maskx = word < 4
    maskd = word == 4
    zero16 = jnp.zeros((16,), F32)
    wordc = jnp.minimum(word, 3)
    cpn = rpt // PCHUNKS

    def _prologue(u_hbm):
        @pl.loop(0, PCHUNKS)
        def _(p):
            rp = r0 + p * cpn
            pltpu.sync_copy(degp.at[0, pl.ds(rp, cpn)], dbuf.at[0])
            pltpu.sync_copy(degp.at[1, pl.ds(rp, cpn)], dbuf.at[1])
            pltpu.sync_copy(xflat.at[pl.ds(rp * 4, cpn * 4)], xbuf)

            @pl.loop(0, cpn // 16)
            def _(g):
                d = dbuf[0, pl.ds(g * 16, 16)] + dbuf[1, pl.ds(g * 16, 16)] + 1.0
                disbuf[pl.ds(g * 16, 16)] = _rsqrt16(d)

            @pl.loop(0, cpn // 2)
            def _(m):
                node = 2 * m + lane8
                dg = plsc.load_gather(disbuf, [node])
                xg = plsc.load_gather(xbuf, [node * 4 + wordc])
                uv = jnp.where(maskx, xg * dg, jnp.where(maskd, dg, zero16))
                plsc.store_scatter(ubuf, [node, word], uv)

            pltpu.sync_copy(ubuf, u_hbm.at[pl.ds(rp, cpn)])

    base = wid * (ept // SUB)

    def _edge_loop(u_hbm):
        _prologue(u_hbm)
        plsc.subcore_barrier()

        @pl.loop(0, ept // CH)
        def _(k):
            p = lax.rem(k, 2)
            pc = p * CH
            pi = p * NSUB

            @pl.when(k >= 2)
            def _():
                for j in range(NSUB):
                    pltpu.make_async_copy(
                        gbuf.at[pl.ds(pc + j * SUB, SUB)],
                        acc.at[idx_c.at[pi + j]], sem_s).wait()

            off = base + k * NSUB
            pltpu.sync_copy(row2d.at[pl.ds(off, NSUB)], idx_r)
            pltpu.sync_copy(col2d.at[pl.ds(off, NSUB)],
                            idx_c.at[pl.ds(pi, NSUB)])
            gd = [pltpu.async_copy(u_hbm.at[idx_r.at[j]],
                                   gbuf.at[pl.ds(pc + j * SUB, SUB)], sem_g)
                  for j in range(NSUB)]
            for d in gd:
                d.wait()
            for j in range(NSUB):
                pltpu.async_copy(gbuf.at[pl.ds(pc + j * SUB, SUB)],
                                 acc.at[idx_c.at[pi + j]], sem_s, add=True)

        for j in range(2 * NSUB):
            pltpu.make_async_copy(gbuf.at[pl.ds(0, SUB)],
                                  acc.at[idx_c.at[0]], sem_s).wait()

    @pl.when(c == 0)
    def _():
        _edge_loop(u0)

    @pl.when(c == 1)
    def _():
        _edge_loop(u1)

    plsc.subcore_barrier()
    pltpu.sync_copy(acc.at[pl.ds(r0, rpt)], out.at[c, pl.ds(r0, rpt)])


def _prop_body(n_pad, ept, u_hbm, row2d, col2d, zeros8, out,
               acc, idx_r, idx_c, gbuf, sem_g, sem_s):
    c = lax.axis_index("c")
    s = lax.axis_index("s")
    wid = c * NS + s
    rpt = n_pad // NS
    pltpu.sync_copy(zeros8.at[pl.ds(s * rpt, rpt)], acc.at[pl.ds(s * rpt, rpt)])
    plsc.subcore_barrier()
    base = wid * (ept // SUB)

    @pl.loop(0, ept // CH)
    def _(k):
        p = lax.rem(k, 2)
        pc = p * CH
        pi = p * NSUB

        @pl.when(k >= 2)
        def _():
            for j in range(NSUB):
                pltpu.make_async_copy(gbuf.at[pl.ds(pc + j * SUB, SUB)],
                                      acc.at[idx_c.at[pi + j]], sem_s).wait()

        off = base + k * NSUB
        pltpu.sync_copy(row2d.at[pl.ds(off, NSUB)], idx_r)
        pltpu.sync_copy(col2d.at[pl.ds(off, NSUB)], idx_c.at[pl.ds(pi, NSUB)])
        gd = [pltpu.async_copy(u_hbm.at[idx_r.at[j]],
                               gbuf.at[pl.ds(pc + j * SUB, SUB)], sem_g)
              for j in range(NSUB)]
        for d in gd:
            d.wait()
        for j in range(NSUB):
            pltpu.async_copy(gbuf.at[pl.ds(pc + j * SUB, SUB)],
                             acc.at[idx_c.at[pi + j]], sem_s, add=True)

    for j in range(2 * NSUB):
        pltpu.make_async_copy(gbuf.at[pl.ds(0, SUB)],
                              acc.at[idx_c.at[0]], sem_s).wait()

    plsc.subcore_barrier()
    pltpu.sync_copy(acc.at[pl.ds(s * rpt, rpt)], out.at[c, pl.ds(s * rpt, rpt)])


def _tc_a(degp_ref, x_ref, u_ref):
    deg = degp_ref[0] + degp_ref[1] + 1.0
    dis = lax.rsqrt(deg)
    xb = x_ref[...]
    ones = jnp.ones((xb.shape[0], 1), F32)
    zeros = jnp.zeros((xb.shape[0], FW - 5), F32)
    u_ref[...] = dis * jnp.concatenate([xb, ones, zeros], axis=1)


def _tc_b(raw1_ref, u_ref, w1aug_ref, w2_ref, wf_ref, u2_ref):
    u = u_ref[...]
    rawsum = raw1_ref[0] + raw1_ref[1] + u
    dis = u[:, 4:5]
    sc = dis * rawsum[:, 4:5]
    pre = dis * lax.dot_general(rawsum, w1aug_ref[...], (((1,), (0,)), ((), ())),
                                preferred_element_type=F32)
    h = jnp.maximum(pre, 0.0)
    t = lax.dot_general(h, w2_ref[...], (((1,), (1,)), ((), ())),
                        preferred_element_type=F32)
    z = lax.dot_general(t, wf_ref[...], (((1,), (1,)), ((), ())),
                        preferred_element_type=F32)
    zeros = jnp.zeros((z.shape[0], FW2 - 4), F32)
    u2_ref[...] = jnp.concatenate([dis * z, sc, dis, zeros], axis=1)


def _tc_c(raw2_ref, u2_ref, wf_ref, b2_ref, bf_ref, out_ref):
    u2 = u2_ref[...]
    dis = u2[:, 3:4]
    sc = u2[:, 2:3]
    rawsum = raw2_ref[0][:, :2] + raw2_ref[1][:, :2] + u2[:, :2]
    cv = lax.dot_general(b2_ref[...], wf_ref[...], (((1,), (1,)), ((), ())),
                         preferred_element_type=F32)
    out_ref[...] = dis * rawsum + sc * cv + bf_ref[...]


def kernel(x, edge_index, W1, b1, W2, b2, Wf, bf):
    n = x.shape[0]
    e = edge_index.shape[1]
    n_pad = _ceil_to(n + 1, max(BN, NS))
    ept = _ceil_to(-(-e // NW), CH)
    e_pad = ept * NW

    ei = edge_index.astype(jnp.int32)
    pad = jnp.full((e_pad - e,), n, dtype=jnp.int32)
    row2d = jnp.concatenate([ei[0], pad]).reshape(e_pad // SUB, SUB)
    col2d = jnp.concatenate([ei[1], pad]).reshape(e_pad // SUB, SUB)
    xpad = jnp.zeros((n_pad, 4), F32).at[:n].set(x)
    zeros1 = jnp.zeros((n_pad,), F32)
    zeros8 = jnp.zeros((n_pad, FW), F32)
    zeros2 = jnp.zeros((n_pad, FW2), F32)
    w1aug = jnp.concatenate([W1.T, b1[None, :], jnp.zeros((FW - 5, 128), F32)],
                            axis=0)

    mesh = _sc_mesh()
    sc_params = pltpu.CompilerParams(use_tc_tiling_on_sc=False,
                                     skip_device_barrier=True)
    sc_params_nl = pltpu.CompilerParams(use_tc_tiling_on_sc=False,
                                        skip_device_barrier=True,
                                        needs_layout_passes=False)
    tc_params = pltpu.CompilerParams(skip_device_barrier=True)

    deg_call = pl.kernel(
        functools.partial(_deg_body, n_pad, ept),
        out_type=jax.ShapeDtypeStruct((NC, n_pad), F32),
        mesh=mesh,
        compiler_params=sc_params,
        scratch_types=[
            pltpu.VMEM_SHARED((n_pad,), F32),
            pltpu.VMEM((NSUB, SUB), jnp.int32),
            pltpu.VMEM((SUB,), F32),
            pltpu.SemaphoreType.DMA,
        ],
    )
    degp = deg_call(row2d, zeros1)

    def _make_prop(fw):
        return pl.kernel(
            functools.partial(_prop_body, n_pad, ept),
            out_type=jax.ShapeDtypeStruct((NC, n_pad, fw), F32),
            mesh=mesh,
            compiler_params=sc_params,
            scratch_types=[
                pltpu.VMEM_SHARED((n_pad, fw), F32),
                pltpu.VMEM((NSUB, SUB), jnp.int32),
                pltpu.VMEM((2 * NSUB, SUB), jnp.int32),
                pltpu.VMEM((2 * CH, fw), F32),
                pltpu.SemaphoreType.DMA,
                pltpu.SemaphoreType.DMA,
            ],
        )

    prop2_call = _make_prop(FW2)

    rpt = n_pad // NS
    prop1_call = pl.kernel(
        functools.partial(_prop1_body, n_pad, ept),
        out_type=[
            jax.ShapeDtypeStruct((n_pad, FW), F32),
            jax.ShapeDtypeStruct((n_pad, FW), F32),
            jax.ShapeDtypeStruct((NC, n_pad, FW), F32),
        ],
        mesh=mesh,
        compiler_params=sc_params_nl,
        scratch_types=[
            pltpu.VMEM_SHARED((n_pad, FW), F32),
            pltpu.VMEM((NSUB, SUB), jnp.int32),
            pltpu.VMEM((2 * NSUB, SUB), jnp.int32),
            pltpu.VMEM((2 * CH, FW), F32),
            pltpu.VMEM((2, rpt // PCHUNKS), F32),
            pltpu.VMEM((rpt // PCHUNKS * 4,), F32),
            pltpu.VMEM((rpt // PCHUNKS, FW), F32),
            pltpu.VMEM((rpt // PCHUNKS,), F32),
            pltpu.SemaphoreType.DMA,
            pltpu.SemaphoreType.DMA,
        ],
    )

    nb = n_pad // BN
    u, _u1, raw1 = prop1_call(degp, xpad.reshape(n_pad * 4), row2d, col2d,
                              zeros8)

    wspec = lambda shp: pl.BlockSpec(shp, lambda i: tuple(0 for _ in shp))
    u2 = pl.pallas_call(
        _tc_b,
        grid=(nb,),
        in_specs=[
            pl.BlockSpec((NC, BN, FW), lambda i: (0, i, 0)),
            pl.BlockSpec((BN, FW), lambda i: (i, 0)),
            wspec((FW, 128)),
            wspec((128, 128)),
            wspec((2, 128)),
        ],
        out_specs=pl.BlockSpec((BN, FW2), lambda i: (i, 0)),
        out_shape=jax.ShapeDtypeStruct((n_pad, FW2), F32),
        compiler_params=tc_params,
    )(raw1, u, w1aug, W2, Wf)

    raw2 = prop2_call(u2, row2d, col2d, zeros2)

    outp = pl.pallas_call(
        _tc_c,
        grid=(nb,),
        in_specs=[
            pl.BlockSpec((NC, BN, FW2), lambda i: (0, i, 0)),
            pl.BlockSpec((BN, FW2), lambda i: (i, 0)),
            wspec((2, 128)),
            wspec((1, 128)),
            wspec((1, 2)),
        ],
        out_specs=pl.BlockSpec((BN, 2), lambda i: (i, 0)),
        out_shape=jax.ShapeDtypeStruct((n_pad, 2), F32),
        compiler_params=tc_params,
    )(raw2, u2, Wf, b2.reshape(1, 128), bf.reshape(1, 2))

    return outp[:n]
